# baseline (speedup 1.0000x reference)
import functools

import jax
import jax.numpy as jnp
from jax import lax
from jax.experimental import pallas as pl
from jax.experimental.pallas import tpu as pltpu
from jax.experimental.pallas import tpu_sc as plsc

NC = 2
NS = 16
NW = NC * NS
LANES = 16
CHUNK = 128


def _sc_scatter(table, src2, dst2, cwa, cwb):
    npad, d = table.shape
    npt = npad // NS
    nbuf = 3
    assert cwa % nbuf == 0 and cwb % nbuf == 0
    cmax = max(cwa, cwb)

    def body(tab_hbm, src_hbm, dst_hbm, out_hbm,
             srcall_v, dstall_v, rows_v, acc_sh, gsems):
        cid = lax.axis_index("c")
        sid = lax.axis_index("s")

        zvec = jnp.zeros((LANES,), jnp.float32)

        def zrow(i, _):
            for j in range(d // LANES):
                rows_v[0][i, pl.ds(j * LANES, LANES)] = zvec
            return _

        lax.fori_loop(0, CHUNK, zrow, 0)
        for r in range(npt // CHUNK):
            pltpu.sync_copy(rows_v[0],
                            acc_sh.at[pl.ds(sid * npt + r * CHUNK, CHUNK)])

        def run(cw, base):
            pltpu.sync_copy(src_hbm.at[pl.ds(base, cw)],
                            srcall_v.at[pl.ds(0, cw)])
            pltpu.sync_copy(dst_hbm.at[pl.ds(base, cw)],
                            dstall_v.at[pl.ds(0, cw)])
            plsc.subcore_barrier()

            for b in range(nbuf):
                pltpu.async_copy(tab_hbm.at[srcall_v.at[b]], rows_v[b],
                                 gsems[b])

            def group(g, carry):
                j0 = g * nbuf
                for b in range(nbuf):
                    j = j0 + b
                    pltpu.make_async_copy(tab_hbm.at[srcall_v.at[j]],
                                          rows_v[b], gsems[b]).wait()
                    pltpu.sync_copy(rows_v[b], acc_sh.at[dstall_v.at[j]],
                                    add=True)

                    @pl.when(j + nbuf < cw)
                    def _prefetch(jj=j + nbuf, bb=b):
                        pltpu.async_copy(tab_hbm.at[srcall_v.at[jj]],
                                         rows_v[bb], gsems[bb])
                return carry

            lax.fori_loop(0, cw // nbuf, group, 0)

        @pl.when(cid == 0)
        def _core0():
            run(cwa, sid * cwa)

        @pl.when(cid == 1)
        def _core1():
            run(cwb, NS * cwa + sid * cwb)

        plsc.subcore_barrier()

        pltpu.sync_copy(acc_sh.at[pl.ds(sid * npt, npt)],
                        out_hbm.at[cid, pl.ds(sid * npt, npt)])

    mesh = plsc.VectorSubcoreMesh(core_axis_name="c", subcore_axis_name="s")
    return pl.kernel(
        body,
        out_type=jax.ShapeDtypeStruct((NC, npad, d), jnp.float32),
        mesh=mesh,
        scratch_types=[
            pltpu.VMEM((cmax, CHUNK), jnp.int32),
            pltpu.VMEM((cmax, CHUNK), jnp.int32),
            [pltpu.VMEM((CHUNK, d), jnp.float32) for _ in range(nbuf)],
            pltpu.VMEM_SHARED((npad, d), jnp.float32),
            [pltpu.SemaphoreType.DMA for _ in range(nbuf)],
        ],
        compiler_params=pltpu.CompilerParams(use_tc_tiling_on_sc=False),
        name=f"gcn_sc_scatter_d{d}",
    )(table, src2, dst2)


def _sc_degree(dst2, cw, npad):
    def body(dst_hbm, out_hbm, dstall_v, deg_v):
        cid = lax.axis_index("c")
        sid = lax.axis_index("s")
        w = cid * NS + sid
        zvec = jnp.zeros((LANES,), jnp.float32)

        def zi(i, carry):
            deg_v[pl.ds(i * LANES, LANES)] = zvec
            return carry

        lax.fori_loop(0, npad // LANES, zi, 0)
        pltpu.sync_copy(dst_hbm.at[pl.ds(w * cw, cw)], dstall_v)
        ones = jnp.ones((LANES,), jnp.float32)

        def row(j, carry):
            for k in range(CHUNK // LANES):
                idx = dstall_v[j, pl.ds(k * LANES, LANES)]
                plsc.addupdate_scatter(deg_v, [idx], ones)
            return carry

        lax.fori_loop(0, cw, row, 0)
        pltpu.sync_copy(deg_v, out_hbm.at[cid, sid])

    mesh = plsc.VectorSubcoreMesh(core_axis_name="c", subcore_axis_name="s")
    return pl.kernel(
        body,
        out_type=jax.ShapeDtypeStruct((NC, NS, npad), jnp.float32),
        mesh=mesh,
        scratch_types=[
            pltpu.VMEM((cw, CHUNK), jnp.int32),
            pltpu.VMEM((npad,), jnp.float32),
        ],
        compiler_params=pltpu.CompilerParams(use_tc_tiling_on_sc=False,
                                             needs_layout_passes=False),
        name="gcn_sc_degree",
    )(dst2)


def _tc_stage_a(degp, xp, w1):
    def body(deg_ref, x_ref, w_ref, dinv_ref, g_ref):
        deg = jnp.sum(deg_ref[...], axis=1, keepdims=True)
        dinv = jnp.where(deg > 0.0, lax.rsqrt(deg), 0.0)
        dinv64 = jnp.broadcast_to(dinv, (deg.shape[0], 64))
        dinv_ref[...] = dinv64
        h = jnp.dot(x_ref[...], w_ref[...], preferred_element_type=jnp.float32)
        g_ref[...] = h * dinv64

    npad = xp.shape[0]
    return pl.pallas_call(
        body,
        out_shape=[jax.ShapeDtypeStruct((npad, 64), jnp.float32),
                   jax.ShapeDtypeStruct((npad, 64), jnp.float32)],
    )(degp, xp, w1)


def _tc_stage_b(p, dinv64, b, w_next):
    def body(p_ref, dinv_ref, b_ref, w_ref, g_ref):
        dinv = dinv_ref[...]
        t = (p_ref[0] + p_ref[1]) * dinv + b_ref[...]
        h = jnp.maximum(t, 0.0)
        g_ref[...] = jnp.dot(h, w_ref[...],
                             preferred_element_type=jnp.float32) * dinv

    npad = dinv64.shape[0]
    return pl.pallas_call(
        body,
        out_shape=jax.ShapeDtypeStruct((npad, 64), jnp.float32),
    )(p, dinv64, b, w_next)


def _tc_head(p, dinv64, b3, wh1, bh1, wh2, bh2):
    def body(p_ref, dinv_ref, b3_ref, wh1_ref, bh1_ref, wh2_ref, bh2_ref,
             o_ref):
        dinv = dinv_ref[...]
        h = jnp.maximum((p_ref[0] + p_ref[1]) * dinv + b3_ref[...], 0.0)
        h = jnp.maximum(
            jnp.dot(h, wh1_ref[...], preferred_element_type=jnp.float32)
            + bh1_ref[...], 0.0)
        o_ref[...] = jnp.dot(h, wh2_ref[...],
                             preferred_element_type=jnp.float32) + bh2_ref[...]

    npad = dinv64.shape[0]
    return pl.pallas_call(
        body,
        out_shape=jax.ShapeDtypeStruct((npad, 1), jnp.float32),
    )(p, dinv64, b3, wh1, bh1, wh2, bh2)


def kernel(x, edge_index, W1, b1, W2, b2, W3, b3, Wh1, bh1, Wh2, bh2):
    n, in_ch = x.shape
    e = edge_index.shape[1]

    ei = edge_index.astype(jnp.int32)
    loops = jnp.arange(n, dtype=jnp.int32)
    src = jnp.concatenate([ei[0], loops])
    dst = jnp.concatenate([ei[1], loops])
    e_tot = e + n
    ctot = -(-e_tot // (NS * CHUNK))
    ctot = -(-ctot // 6) * 6
    cwa = -(-(ctot * 11) // (18 * 3)) * 3
    cwb = ctot - cwa
    e_pad = NS * ctot * CHUNK
    src = jnp.concatenate([src, jnp.zeros((e_pad - e_tot,), jnp.int32)])
    dst = jnp.concatenate([dst, jnp.full((e_pad - e_tot,), n, jnp.int32)])
    src2 = src.reshape(-1, CHUNK)
    dst2 = dst.reshape(-1, CHUNK)
    cw_d = ctot // 2

    npad = -(-(n + 1) // (NS * CHUNK)) * (NS * CHUNK)

    degp = _sc_degree(dst2, cw_d, npad)
    degt = degp.reshape(NW, npad).T

    xp = jnp.pad(x, ((0, npad - n), (0, 0)))
    dinv64, g1 = _tc_stage_a(degt, xp, W1)

    p1 = _sc_scatter(g1, src2, dst2, cwa, cwb)
    g2 = _tc_stage_b(p1, dinv64, b1.reshape(1, -1), W2)
    p2 = _sc_scatter(g2, src2, dst2, cwa, cwb)
    g3 = _tc_stage_b(p2, dinv64, b2.reshape(1, -1), W3)
    p3 = _sc_scatter(g3, src2, dst2, cwa, cwb)

    out = _tc_head(p3, dinv64, b3.reshape(1, -1), Wh1, bh1.reshape(1, -1),
                   Wh2, bh2.reshape(1, 1))
    return out[:n, 0]


# speedup vs baseline: 1.0119x; 1.0119x over previous
import functools

import jax
import jax.numpy as jnp
from jax import lax
from jax.experimental import pallas as pl
from jax.experimental.pallas import tpu as pltpu
from jax.experimental.pallas import tpu_sc as plsc

NC = 2
NS = 16
NW = NC * NS
LANES = 16
CHUNK = 128


def _sc_scatter(table, src2, dst2, cwa, cwb):
    npad, d = table.shape
    npt = npad // NS
    nbuf = 3
    assert cwa % nbuf == 0 and cwb % nbuf == 0
    cmax = max(cwa, cwb)

    def body(tab_hbm, src_hbm, dst_hbm, out_hbm,
             srcall_v, dstall_v, rows_v, zero_v, acc_sh, gsems):
        cid = lax.axis_index("c")
        sid = lax.axis_index("s")

        def run(cw, base):
            pltpu.sync_copy(src_hbm.at[pl.ds(base, cw)],
                            srcall_v.at[pl.ds(0, cw)])
            pltpu.sync_copy(dst_hbm.at[pl.ds(base, cw)],
                            dstall_v.at[pl.ds(0, cw)])
            for b in range(nbuf):
                pltpu.async_copy(tab_hbm.at[srcall_v.at[b]], rows_v[b],
                                 gsems[b])

            zvec = jnp.zeros((LANES,), jnp.float32)

            def zrow(i, carry):
                for jz in range(d // LANES):
                    zero_v[i, pl.ds(jz * LANES, LANES)] = zvec
                return carry

            lax.fori_loop(0, CHUNK, zrow, 0)
            for r in range(npt // CHUNK):
                pltpu.sync_copy(zero_v,
                                acc_sh.at[pl.ds(sid * npt + r * CHUNK,
                                                CHUNK)])
            plsc.subcore_barrier()

            def group(g, carry):
                j0 = g * nbuf
                for b in range(nbuf):
                    j = j0 + b
                    pltpu.make_async_copy(tab_hbm.at[srcall_v.at[j]],
                                          rows_v[b], gsems[b]).wait()
                    pltpu.sync_copy(rows_v[b], acc_sh.at[dstall_v.at[j]],
                                    add=True)

                    @pl.when(j + nbuf < cw)
                    def _prefetch(jj=j + nbuf, bb=b):
                        pltpu.async_copy(tab_hbm.at[srcall_v.at[jj]],
                                         rows_v[bb], gsems[bb])
                return carry

            lax.fori_loop(0, cw // nbuf, group, 0)

        @pl.when(cid == 0)
        def _core0():
            run(cwa, sid * cwa)

        @pl.when(cid == 1)
        def _core1():
            run(cwb, NS * cwa + sid * cwb)

        plsc.subcore_barrier()

        pltpu.sync_copy(acc_sh.at[pl.ds(sid * npt, npt)],
                        out_hbm.at[cid, pl.ds(sid * npt, npt)])

    mesh = plsc.VectorSubcoreMesh(core_axis_name="c", subcore_axis_name="s")
    return pl.kernel(
        body,
        out_type=jax.ShapeDtypeStruct((NC, npad, d), jnp.float32),
        mesh=mesh,
        scratch_types=[
            pltpu.VMEM((cmax, CHUNK), jnp.int32),
            pltpu.VMEM((cmax, CHUNK), jnp.int32),
            [pltpu.VMEM((CHUNK, d), jnp.float32) for _ in range(nbuf)],
            pltpu.VMEM((CHUNK, d), jnp.float32),
            pltpu.VMEM_SHARED((npad, d), jnp.float32),
            [pltpu.SemaphoreType.DMA for _ in range(nbuf)],
        ],
        compiler_params=pltpu.CompilerParams(use_tc_tiling_on_sc=False),
        name=f"gcn_sc_scatter_d{d}",
    )(table, src2, dst2)


def _sc_degree(dst2, cw, npad):
    def body(dst_hbm, out_hbm, dstall_v, deg_v):
        cid = lax.axis_index("c")
        sid = lax.axis_index("s")
        w = cid * NS + sid
        zvec = jnp.zeros((LANES,), jnp.float32)

        def zi(i, carry):
            deg_v[pl.ds(i * LANES, LANES)] = zvec
            return carry

        lax.fori_loop(0, npad // LANES, zi, 0)
        pltpu.sync_copy(dst_hbm.at[pl.ds(w * cw, cw)], dstall_v)
        ones = jnp.ones((LANES,), jnp.float32)

        def row(j, carry):
            for k in range(CHUNK // LANES):
                idx = dstall_v[j, pl.ds(k * LANES, LANES)]
                plsc.addupdate_scatter(deg_v, [idx], ones)
            return carry

        lax.fori_loop(0, cw, row, 0)
        pltpu.sync_copy(deg_v, out_hbm.at[cid, sid])

    mesh = plsc.VectorSubcoreMesh(core_axis_name="c", subcore_axis_name="s")
    return pl.kernel(
        body,
        out_type=jax.ShapeDtypeStruct((NC, NS, npad), jnp.float32),
        mesh=mesh,
        scratch_types=[
            pltpu.VMEM((cw, CHUNK), jnp.int32),
            pltpu.VMEM((npad,), jnp.float32),
        ],
        compiler_params=pltpu.CompilerParams(use_tc_tiling_on_sc=False,
                                             needs_layout_passes=False),
        name="gcn_sc_degree",
    )(dst2)


def _tc_stage_a(degp, xp, w1):
    def body(deg_ref, x_ref, w_ref, dinv_ref, g_ref):
        deg = jnp.sum(deg_ref[...], axis=1, keepdims=True)
        dinv = jnp.where(deg > 0.0, lax.rsqrt(deg), 0.0)
        dinv64 = jnp.broadcast_to(dinv, (deg.shape[0], 64))
        dinv_ref[...] = dinv64
        h = jnp.dot(x_ref[...], w_ref[...], preferred_element_type=jnp.float32)
        g_ref[...] = h * dinv64

    npad = xp.shape[0]
    return pl.pallas_call(
        body,
        out_shape=[jax.ShapeDtypeStruct((npad, 64), jnp.float32),
                   jax.ShapeDtypeStruct((npad, 64), jnp.float32)],
    )(degp, xp, w1)


def _tc_stage_b(p, dinv64, b, w_next):
    def body(p_ref, dinv_ref, b_ref, w_ref, g_ref):
        dinv = dinv_ref[...]
        t = (p_ref[0] + p_ref[1]) * dinv + b_ref[...]
        h = jnp.maximum(t, 0.0)
        g_ref[...] = jnp.dot(h, w_ref[...],
                             preferred_element_type=jnp.float32) * dinv

    npad = dinv64.shape[0]
    return pl.pallas_call(
        body,
        out_shape=jax.ShapeDtypeStruct((npad, 64), jnp.float32),
    )(p, dinv64, b, w_next)


def _tc_head(p, dinv64, b3, wh1, bh1, wh2, bh2):
    def body(p_ref, dinv_ref, b3_ref, wh1_ref, bh1_ref, wh2_ref, bh2_ref,
             o_ref):
        dinv = dinv_ref[...]
        h = jnp.maximum((p_ref[0] + p_ref[1]) * dinv + b3_ref[...], 0.0)
        h = jnp.maximum(
            jnp.dot(h, wh1_ref[...], preferred_element_type=jnp.float32)
            + bh1_ref[...], 0.0)
        o_ref[...] = jnp.dot(h, wh2_ref[...],
                             preferred_element_type=jnp.float32) + bh2_ref[...]

    npad = dinv64.shape[0]
    return pl.pallas_call(
        body,
        out_shape=jax.ShapeDtypeStruct((npad, 1), jnp.float32),
    )(p, dinv64, b3, wh1, bh1, wh2, bh2)


def kernel(x, edge_index, W1, b1, W2, b2, W3, b3, Wh1, bh1, Wh2, bh2):
    n, in_ch = x.shape
    e = edge_index.shape[1]

    ei = edge_index.astype(jnp.int32)
    loops = jnp.arange(n, dtype=jnp.int32)
    src = jnp.concatenate([ei[0], loops])
    dst = jnp.concatenate([ei[1], loops])
    e_tot = e + n
    ctot = -(-e_tot // (NS * CHUNK))
    ctot = -(-ctot // 6) * 6
    cwa = -(-(ctot * 11) // (18 * 3)) * 3
    cwb = ctot - cwa
    e_pad = NS * ctot * CHUNK
    src = jnp.concatenate([src, jnp.zeros((e_pad - e_tot,), jnp.int32)])
    dst = jnp.concatenate([dst, jnp.full((e_pad - e_tot,), n, jnp.int32)])
    src2 = src.reshape(-1, CHUNK)
    dst2 = dst.reshape(-1, CHUNK)
    cw_d = ctot // 2

    npad = -(-(n + 1) // (NS * CHUNK)) * (NS * CHUNK)

    degp = _sc_degree(dst2, cw_d, npad)
    degt = degp.reshape(NW, npad).T

    xp = jnp.pad(x, ((0, npad - n), (0, 0)))
    dinv64, g1 = _tc_stage_a(degt, xp, W1)

    p1 = _sc_scatter(g1, src2, dst2, cwa, cwb)
    g2 = _tc_stage_b(p1, dinv64, b1.reshape(1, -1), W2)
    p2 = _sc_scatter(g2, src2, dst2, cwa, cwb)
    g3 = _tc_stage_b(p2, dinv64, b2.reshape(1, -1), W3)
    p3 = _sc_scatter(g3, src2, dst2, cwa, cwb)

    out = _tc_head(p3, dinv64, b3.reshape(1, -1), Wh1, bh1.reshape(1, -1),
                   Wh2, bh2.reshape(1, 1))
    return out[:n, 0]
